# baseline (speedup 1.0000x reference)
import functools

import jax
import jax.numpy as jnp
from jax import lax
from jax.experimental import pallas as pl
from jax.experimental.pallas import tpu as pltpu
from jax.experimental.pallas import tpu_sc as plsc

EMB_DIM = 64
BATCH = 16384
CTX = 20
NW = 32
CB = 16
ROWS_PER_W = 2 * BATCH // NW
CHUNKS = ROWS_PER_W // CB
NIDX = CB * CTX
STREAMS = [(0, 128), (128, 128), (256, 64)]


def _sc_partials(u2, v2, all_u, all_v):
    mesh = plsc.VectorSubcoreMesh(core_axis_name="c", subcore_axis_name="s")

    @functools.partial(
        pl.kernel,
        mesh=mesh,
        compiler_params=pltpu.CompilerParams(use_tc_tiling_on_sc=False),
        out_type=jax.ShapeDtypeStruct((2 * BATCH, 16), jnp.float32),
        scratch_types=[
            pltpu.VMEM((2, CB, 128), jnp.int32),
            pltpu.VMEM((2, NIDX), jnp.int32),
            pltpu.VMEM((2, NIDX + 16), jnp.int32),
            pltpu.VMEM((2, CB), jnp.int32),
            pltpu.VMEM((2, CB + 16), jnp.int32),
            pltpu.VMEM((2, NIDX, 128), jnp.float32),
            pltpu.VMEM((2, CB, 128), jnp.float32),
            pltpu.VMEM((CB, 16), jnp.float32),
            pltpu.SemaphoreType.DMA,
            pltpu.SemaphoreType.DMA,
        ],
    )
    def k(u_hbm, v_hbm, uidx_hbm, vidx_hbm, out_hbm,
          uidx_v, cidx_v, cpar_v, vjid_v, vpar_v, rows_v, vrows_v, part_v,
          sem0, sem1):
        wid = lax.axis_index("s") * 2 + lax.axis_index("c")
        base = wid * ROWS_PER_W
        sems = (sem0, sem1)

        def stage2(ci, bufi):
            r0 = base + ci * CB
            pltpu.sync_copy(uidx_hbm.at[pl.ds(r0, CB)], uidx_v.at[bufi])

            def compact(b, carry):
                o = b * CTX
                x0 = uidx_v[bufi, b, pl.ds(0, 16)]
                x1 = uidx_v[bufi, b, pl.ds(4, 16)]
                cidx_v[bufi, pl.ds(o, 16)] = jnp.right_shift(x0, 1)
                cidx_v[bufi, pl.ds(o + 4, 16)] = jnp.right_shift(x1, 1)
                cpar_v[bufi, pl.ds(o, 16)] = jnp.left_shift(
                    jnp.bitwise_and(x0, 1), 6)
                cpar_v[bufi, pl.ds(o + 4, 16)] = jnp.left_shift(
                    jnp.bitwise_and(x1, 1), 6)
                return carry

            lax.fori_loop(0, CB, compact, 0)

            pltpu.sync_copy(vidx_hbm.at[pl.ds(r0, CB)], vjid_v.at[bufi])
            y = vjid_v[bufi, pl.ds(0, 16)]
            vpar_v[bufi, pl.ds(0, 16)] = jnp.left_shift(
                jnp.bitwise_and(y, 1), 6)
            vjid_v[bufi, pl.ds(0, 16)] = jnp.right_shift(y, 1)

            pltpu.async_copy(v_hbm.at[vjid_v.at[bufi]], vrows_v.at[bufi],
                             sems[bufi])
            for (o, n) in STREAMS:
                pltpu.async_copy(
                    u_hbm.at[cidx_v.at[bufi, pl.ds(o, n)]],
                    rows_v.at[bufi, pl.ds(o, n)], sems[bufi])

        def process(ci, bufi):
            r0 = base + ci * CB
            pltpu.make_async_copy(v_hbm.at[vjid_v.at[bufi]],
                                  vrows_v.at[bufi], sems[bufi]).wait()
            for (o, n) in STREAMS:
                pltpu.make_async_copy(
                    u_hbm.at[cidx_v.at[bufi, pl.ds(o, n)]],
                    rows_v.at[bufi, pl.ds(o, n)], sems[bufi]).wait()

            def row_body(b, carry):
                r = b * CTX
                o0 = cpar_v[bufi, pl.ds(r, 16)][0]
                a0 = rows_v[bufi, r, pl.ds(o0, 16)]
                a1 = rows_v[bufi, r, pl.ds(o0 + 16, 16)]
                a2 = rows_v[bufi, r, pl.ds(o0 + 32, 16)]
                a3 = rows_v[bufi, r, pl.ds(o0 + 48, 16)]
                for c in range(1, CTX):
                    oc = cpar_v[bufi, pl.ds(r + c, 16)][0]
                    a0 = a0 + rows_v[bufi, r + c, pl.ds(oc, 16)]
                    a1 = a1 + rows_v[bufi, r + c, pl.ds(oc + 16, 16)]
                    a2 = a2 + rows_v[bufi, r + c, pl.ds(oc + 32, 16)]
                    a3 = a3 + rows_v[bufi, r + c, pl.ds(oc + 48, 16)]
                ov = vpar_v[bufi, pl.ds(b, 16)][0]
                p = (a0 * vrows_v[bufi, b, pl.ds(ov, 16)]
                     + a1 * vrows_v[bufi, b, pl.ds(ov + 16, 16)]
                     + a2 * vrows_v[bufi, b, pl.ds(ov + 32, 16)]
                     + a3 * vrows_v[bufi, b, pl.ds(ov + 48, 16)])
                part_v[b, :] = p
                return carry

            lax.fori_loop(0, CB, row_body, 0)
            pltpu.sync_copy(part_v.at[pl.ds(0, CB)], out_hbm.at[pl.ds(r0, CB)])

        stage2(0, 0)

        def body2(h, carry):
            ci = 2 * h
            stage2(ci + 1, 1)
            process(ci, 0)

            @pl.when(ci + 2 < CHUNKS)
            def _():
                stage2(ci + 2, 0)

            process(ci + 1, 1)
            return carry

        lax.fori_loop(0, CHUNKS // 2, body2, 0)

    return k(u2, v2, all_u, all_v)


def _tc_loss(partials):
    def body(p_ref, o_ref):
        x = p_ref[...]
        s = jnp.sum(x, axis=1, keepdims=True)
        row = lax.broadcasted_iota(jnp.int32, (2 * BATCH, 1), 0)
        z = jnp.where(row < BATCH, s, -s)
        l = jnp.minimum(z, 0.0) - jnp.log1p(jnp.exp(-jnp.abs(z)))
        o_ref[0, 0] = -jnp.sum(l)

    out = pl.pallas_call(
        body,
        out_shape=jax.ShapeDtypeStruct((1, 1), jnp.float32),
        out_specs=pl.BlockSpec(memory_space=pltpu.SMEM),
    )(partials)
    return out[0, 0]


def kernel(pos_u, pos_v, neg_u, neg_v, u_weight, v_weight):
    u2 = jnp.reshape(u_weight, (u_weight.shape[0] // 2, 2 * EMB_DIM))
    v2 = jnp.reshape(v_weight, (v_weight.shape[0] // 2, 2 * EMB_DIM))
    all_u = jnp.pad(jnp.concatenate([pos_u, neg_u], axis=0),
                    ((0, 0), (0, 128 - CTX)))
    all_v = jnp.concatenate([pos_v, neg_v], axis=0)
    partials = _sc_partials(u2, v2, all_u, all_v)
    return _tc_loss(partials)


# speedup vs baseline: 1.2084x; 1.2084x over previous
import functools

import jax
import jax.numpy as jnp
from jax import lax
from jax.experimental import pallas as pl
from jax.experimental.pallas import tpu as pltpu
from jax.experimental.pallas import tpu_sc as plsc

EMB_DIM = 64
BATCH = 16384
CTX = 20
NW = 32
CB = 32
ROWS_PER_W = 2 * BATCH // NW
CHUNKS = ROWS_PER_W // CB
GPC = CB * CTX // 128


def _sc_pool(u_weight, all_u):
    mesh = plsc.VectorSubcoreMesh(core_axis_name="c", subcore_axis_name="s")

    @functools.partial(
        pl.kernel,
        mesh=mesh,
        compiler_params=pltpu.CompilerParams(use_tc_tiling_on_sc=False),
        out_type=jax.ShapeDtypeStruct((2 * BATCH, EMB_DIM), jnp.float32),
        scratch_types=[
            pltpu.VMEM((2, CB, 128), jnp.int32),
            pltpu.VMEM((2, CB * CTX), jnp.int32),
            pltpu.VMEM((2, CB * CTX, EMB_DIM), jnp.float32),
            pltpu.VMEM((CB, EMB_DIM), jnp.float32),
            pltpu.SemaphoreType.DMA,
            pltpu.SemaphoreType.DMA,
        ],
    )
    def k(u_hbm, uidx_hbm, out_hbm, uidx_v, cidx_v, rows_v, pool_v,
          sem0, sem1):
        wid = lax.axis_index("s") * 2 + lax.axis_index("c")
        base = wid * ROWS_PER_W
        sems = (sem0, sem1)

        def stage(ci, bufi):
            r0 = base + ci * CB
            pltpu.sync_copy(uidx_hbm.at[pl.ds(r0, CB)], uidx_v.at[bufi])

            def compact(b, carry):
                o = b * CTX
                cidx_v[bufi, pl.ds(o, 16)] = uidx_v[bufi, b, pl.ds(0, 16)]
                cidx_v[bufi, pl.ds(o + 4, 16)] = uidx_v[bufi, b, pl.ds(4, 16)]
                return carry

            lax.fori_loop(0, CB, compact, 0)
            for j in range(GPC):
                pltpu.async_copy(
                    u_hbm.at[cidx_v.at[bufi, pl.ds(j * 128, 128)]],
                    rows_v.at[bufi, pl.ds(j * 128, 128)], sems[bufi])

        def process(ci, bufi):
            r0 = base + ci * CB
            for j in range(GPC):
                pltpu.make_async_copy(
                    u_hbm.at[cidx_v.at[bufi, pl.ds(j * 128, 128)]],
                    rows_v.at[bufi, pl.ds(j * 128, 128)], sems[bufi]).wait()

            def row_body(b, carry):
                r = b * CTX
                a0 = rows_v[bufi, r, pl.ds(0, 16)]
                a1 = rows_v[bufi, r, pl.ds(16, 16)]
                a2 = rows_v[bufi, r, pl.ds(32, 16)]
                a3 = rows_v[bufi, r, pl.ds(48, 16)]
                for c in range(1, CTX):
                    a0 = a0 + rows_v[bufi, r + c, pl.ds(0, 16)]
                    a1 = a1 + rows_v[bufi, r + c, pl.ds(16, 16)]
                    a2 = a2 + rows_v[bufi, r + c, pl.ds(32, 16)]
                    a3 = a3 + rows_v[bufi, r + c, pl.ds(48, 16)]
                pool_v[b, pl.ds(0, 16)] = a0
                pool_v[b, pl.ds(16, 16)] = a1
                pool_v[b, pl.ds(32, 16)] = a2
                pool_v[b, pl.ds(48, 16)] = a3
                return carry

            lax.fori_loop(0, CB, row_body, 0)
            pltpu.sync_copy(pool_v, out_hbm.at[pl.ds(r0, CB)])

        stage(0, 0)

        def body2(h, carry):
            ci = 2 * h
            stage(ci + 1, 1)
            process(ci, 0)

            @pl.when(ci + 2 < CHUNKS)
            def _():
                stage(ci + 2, 0)

            process(ci + 1, 1)
            return carry

        lax.fori_loop(0, CHUNKS // 2, body2, 0)

    return k(u_weight, all_u)


def _sc_dot(v_weight, all_v, pooled):
    mesh = plsc.VectorSubcoreMesh(core_axis_name="c", subcore_axis_name="s")

    @functools.partial(
        pl.kernel,
        mesh=mesh,
        out_type=jax.ShapeDtypeStruct((2 * BATCH, 16), jnp.float32),
        scratch_types=[
            pltpu.VMEM((2, CB + 16), jnp.int32),
            pltpu.VMEM((2, CB * 8, EMB_DIM), jnp.float32),
            pltpu.VMEM((2, CB, EMB_DIM), jnp.float32),
            pltpu.VMEM((CB, 16), jnp.float32),
            pltpu.SemaphoreType.DMA,
            pltpu.SemaphoreType.DMA,
        ],
    )
    def k(v_hbm, vidx_hbm, pool_hbm, out_hbm, vidx_v, slab_v, pool_v, part_v,
          sem0, sem1):
        wid = lax.axis_index("s") * 2 + lax.axis_index("c")
        base = wid * ROWS_PER_W
        sems = (sem0, sem1)

        def stage(ci, bufi):
            r0 = base + ci * CB
            pltpu.sync_copy(vidx_hbm.at[pl.ds(r0, CB)],
                            vidx_v.at[bufi, pl.ds(0, CB)])
            pltpu.async_copy(pool_hbm.at[pl.ds(r0, CB)], pool_v.at[bufi],
                             sems[bufi])
            for b in range(CB):
                i = vidx_v[bufi, pl.ds(b, 16)][0]
                j0 = pl.multiple_of((i >> 3) * 8, 8)
                pltpu.async_copy(v_hbm.at[pl.ds(j0, 8)],
                                 slab_v.at[bufi, pl.ds(b * 8, 8)], sems[bufi])

        def process(ci, bufi):
            r0 = base + ci * CB
            pltpu.make_async_copy(pool_hbm.at[pl.ds(r0, CB)], pool_v.at[bufi],
                                  sems[bufi]).wait()
            for b in range(CB):
                pltpu.make_async_copy(
                    v_hbm.at[pl.ds(0, 8)],
                    slab_v.at[bufi, pl.ds(b * 8, 8)], sems[bufi]).wait()

            def row_body(b, carry):
                s = jnp.bitwise_and(vidx_v[bufi, pl.ds(b, 16)][0], 7)
                r = b * 8 + s
                p = (pool_v[bufi, b, pl.ds(0, 16)]
                     * slab_v[bufi, r, pl.ds(0, 16)]
                     + pool_v[bufi, b, pl.ds(16, 16)]
                     * slab_v[bufi, r, pl.ds(16, 16)]
                     + pool_v[bufi, b, pl.ds(32, 16)]
                     * slab_v[bufi, r, pl.ds(32, 16)]
                     + pool_v[bufi, b, pl.ds(48, 16)]
                     * slab_v[bufi, r, pl.ds(48, 16)])
                part_v[b, :] = p
                return carry

            lax.fori_loop(0, CB, row_body, 0)
            pltpu.sync_copy(part_v, out_hbm.at[pl.ds(r0, CB)])

        stage(0, 0)

        def body2(h, carry):
            ci = 2 * h
            stage(ci + 1, 1)
            process(ci, 0)

            @pl.when(ci + 2 < CHUNKS)
            def _():
                stage(ci + 2, 0)

            process(ci + 1, 1)
            return carry

        lax.fori_loop(0, CHUNKS // 2, body2, 0)

    return k(v_weight, all_v, pooled)


def _tc_loss(partials):
    def body(p_ref, o_ref):
        x = p_ref[...]
        s = jnp.sum(x, axis=1, keepdims=True)
        row = lax.broadcasted_iota(jnp.int32, (2 * BATCH, 1), 0)
        z = jnp.where(row < BATCH, s, -s)
        l = jnp.minimum(z, 0.0) - jnp.log1p(jnp.exp(-jnp.abs(z)))
        o_ref[0, 0] = -jnp.sum(l)

    out = pl.pallas_call(
        body,
        out_shape=jax.ShapeDtypeStruct((1, 1), jnp.float32),
        out_specs=pl.BlockSpec(memory_space=pltpu.SMEM),
    )(partials)
    return out[0, 0]


def kernel(pos_u, pos_v, neg_u, neg_v, u_weight, v_weight):
    all_u = jnp.pad(jnp.concatenate([pos_u, neg_u], axis=0),
                    ((0, 0), (0, 128 - CTX)))
    all_v = jnp.concatenate([pos_v, neg_v], axis=0)
    pooled = _sc_pool(u_weight, all_u)
    partials = _sc_dot(v_weight, all_v, pooled)
    return _tc_loss(partials)
